# baseline (device time: 124304 ns/iter reference)
import jax
import jax.numpy as jnp
from jax import lax
from jax.experimental import pallas as pl
from jax.experimental.pallas import tpu as pltpu

N_DEV = 16
B, SQ, SKV, HQ_LOC, DH = 2, 128, 128, 4, 64
D_MODEL = 512
D_HEADS = HQ_LOC * DH


def kernel(x, Wq, K_ext, V_ext, Wo):
    my = lax.axis_index("i")
    Wq_my = lax.dynamic_slice_in_dim(Wq, my * D_HEADS, D_HEADS, axis=1)
    Wo_my = lax.dynamic_slice_in_dim(Wo, my * D_HEADS, D_HEADS, axis=0)

    def body(x_ref, wq_ref, k_ref, v_ref, wo_ref, out_ref,
             comm_ref, send_sems, recv_sems):
        my_pos = lax.axis_index("i")
        left = lax.rem(my_pos + N_DEV - 1, N_DEV)
        right = lax.rem(my_pos + 1, N_DEV)

        barrier_sem = pltpu.get_barrier_semaphore()
        for nbr in [left, right]:
            pl.semaphore_signal(
                barrier_sem, inc=1,
                device_id=(nbr,), device_id_type=pl.DeviceIdType.MESH,
            )
        pl.semaphore_wait(barrier_sem, 2)

        xm = x_ref[...].reshape(B * SQ, D_MODEL)
        q = jnp.dot(xm, wq_ref[...], preferred_element_type=jnp.float32)
        q4 = q.reshape(B, SQ, HQ_LOC, DH)

        ctx_rows = []
        for b in range(B):
            parts = []
            for h in range(HQ_LOC):
                qb = q4[b, :, h, :]
                kb = k_ref[b, :, h, :]
                s = lax.dot_general(
                    qb, kb, (((1,), (1,)), ((), ())),
                    preferred_element_type=jnp.float32,
                ) * 0.125
                m = jnp.max(s, axis=-1, keepdims=True)
                w = jnp.exp(s - m)
                w = w / jnp.sum(w, axis=-1, keepdims=True)
                vb = v_ref[b, :, h, :]
                parts.append(
                    jnp.dot(w, vb, preferred_element_type=jnp.float32)
                )
            ctx_rows.append(jnp.concatenate(parts, axis=-1))
        ctx = jnp.concatenate(ctx_rows, axis=0)

        partial = jnp.dot(
            ctx, wo_ref[...], preferred_element_type=jnp.float32
        ).reshape(B, SQ, D_MODEL)

        comm_ref[0] = partial
        acc = partial

        for h in range(N_DEV - 1):
            rdma = pltpu.make_async_remote_copy(
                src_ref=comm_ref.at[h],
                dst_ref=comm_ref.at[h + 1],
                send_sem=send_sems.at[h],
                recv_sem=recv_sems.at[h],
                device_id=(right,),
                device_id_type=pl.DeviceIdType.MESH,
            )
            rdma.start()
            rdma.wait()
            acc = acc + comm_ref[h + 1]

        out_ref[...] = acc

    return pl.pallas_call(
        body,
        out_shape=jax.ShapeDtypeStruct((B, SQ, D_MODEL), jnp.float32),
        in_specs=[
            pl.BlockSpec(memory_space=pltpu.VMEM),
            pl.BlockSpec(memory_space=pltpu.VMEM),
            pl.BlockSpec(memory_space=pltpu.VMEM),
            pl.BlockSpec(memory_space=pltpu.VMEM),
            pl.BlockSpec(memory_space=pltpu.VMEM),
        ],
        out_specs=pl.BlockSpec(memory_space=pltpu.VMEM),
        scratch_shapes=[
            pltpu.VMEM((N_DEV, B, SQ, D_MODEL), jnp.float32),
            pltpu.SemaphoreType.DMA((N_DEV - 1,)),
            pltpu.SemaphoreType.DMA((N_DEV - 1,)),
        ],
        compiler_params=pltpu.CompilerParams(collective_id=0),
    )(x, Wq_my, K_ext, V_ext, Wo_my)


# device time: 38029 ns/iter; 3.2687x vs baseline; 3.2687x over previous
import jax
import jax.numpy as jnp
from jax import lax
from jax.experimental import pallas as pl
from jax.experimental.pallas import tpu as pltpu

N_DEV = 16
B, SQ, SKV, HQ_LOC, DH = 2, 128, 128, 4, 64
D_MODEL = 512
D_HEADS = HQ_LOC * DH
ROWS = B * SQ

RS_BITS = (0, 1, 2, 3)
RS_LEN = (128, 64, 32, 16)
RS_OFF = (0, 128, 192, 224)


def kernel(x, Wq, K_ext, V_ext, Wo):
    my = lax.axis_index("i")
    Wq_my = lax.dynamic_slice_in_dim(Wq, my * D_HEADS, D_HEADS, axis=1)
    Wo_my = lax.dynamic_slice_in_dim(Wo, my * D_HEADS, D_HEADS, axis=0)

    def body(x_ref, wq_ref, k_ref, v_ref, wo_ref, out_ref,
             acc_ref, recv_ref, rs_send, rs_recv, ag_send, ag_recv):
        my_pos = lax.axis_index("i")

        barrier_sem = pltpu.get_barrier_semaphore()
        for b in RS_BITS:
            pl.semaphore_signal(
                barrier_sem, inc=1,
                device_id=(my_pos ^ (1 << b),),
                device_id_type=pl.DeviceIdType.MESH,
            )
        pl.semaphore_wait(barrier_sem, len(RS_BITS))

        xm = x_ref[...].reshape(ROWS, D_MODEL)
        q = jnp.dot(xm, wq_ref[...], preferred_element_type=jnp.float32)
        q4 = q.reshape(B, SQ, HQ_LOC, DH)

        ctx_rows = []
        for bb in range(B):
            parts = []
            for h in range(HQ_LOC):
                qb = q4[bb, :, h, :]
                kb = k_ref[bb, :, h, :]
                s = lax.dot_general(
                    qb, kb, (((1,), (1,)), ((), ())),
                    preferred_element_type=jnp.float32,
                ) * 0.125
                m = jnp.max(s, axis=-1, keepdims=True)
                w = jnp.exp(s - m)
                w = w / jnp.sum(w, axis=-1, keepdims=True)
                vb = v_ref[bb, :, h, :]
                parts.append(
                    jnp.dot(w, vb, preferred_element_type=jnp.float32)
                )
            ctx_rows.append(jnp.concatenate(parts, axis=-1))
        ctx = jnp.concatenate(ctx_rows, axis=0)

        acc_ref[...] = jnp.dot(
            ctx, wo_ref[...], preferred_element_type=jnp.float32
        )

        lo = jnp.int32(0)
        for k, b in enumerate(RS_BITS):
            n = RS_LEN[k]
            partner = my_pos ^ (1 << b)
            mybit = (my_pos >> b) & 1
            keep_lo = lo + mybit * n
            send_lo = lo + (1 - mybit) * n
            rdma = pltpu.make_async_remote_copy(
                src_ref=acc_ref.at[pl.ds(send_lo, n), :],
                dst_ref=recv_ref.at[pl.ds(RS_OFF[k], n), :],
                send_sem=rs_send.at[k],
                recv_sem=rs_recv.at[k],
                device_id=(partner,),
                device_id_type=pl.DeviceIdType.MESH,
            )
            rdma.start()
            rdma.wait()
            acc_ref[pl.ds(keep_lo, n), :] = (
                acc_ref[pl.ds(keep_lo, n), :]
                + recv_ref[pl.ds(RS_OFF[k], n), :]
            )
            lo = keep_lo

        for k in reversed(range(len(RS_BITS))):
            b = RS_BITS[k]
            n = RS_LEN[k]
            partner = my_pos ^ (1 << b)
            mybit = (my_pos >> b) & 1
            rdma = pltpu.make_async_remote_copy(
                src_ref=acc_ref.at[pl.ds(lo, n), :],
                dst_ref=acc_ref.at[pl.ds(lo, n), :],
                send_sem=ag_send.at[k],
                recv_sem=ag_recv.at[k],
                device_id=(partner,),
                device_id_type=pl.DeviceIdType.MESH,
            )
            rdma.start()
            rdma.wait()
            lo = lo - mybit * n

        out_ref[...] = acc_ref[...].reshape(B, SQ, D_MODEL)

    return pl.pallas_call(
        body,
        out_shape=jax.ShapeDtypeStruct((B, SQ, D_MODEL), jnp.float32),
        in_specs=[
            pl.BlockSpec(memory_space=pltpu.VMEM),
            pl.BlockSpec(memory_space=pltpu.VMEM),
            pl.BlockSpec(memory_space=pltpu.VMEM),
            pl.BlockSpec(memory_space=pltpu.VMEM),
            pl.BlockSpec(memory_space=pltpu.VMEM),
        ],
        out_specs=pl.BlockSpec(memory_space=pltpu.VMEM),
        scratch_shapes=[
            pltpu.VMEM((ROWS, D_MODEL), jnp.float32),
            pltpu.VMEM((240, D_MODEL), jnp.float32),
            pltpu.SemaphoreType.DMA((4,)),
            pltpu.SemaphoreType.DMA((4,)),
            pltpu.SemaphoreType.DMA((4,)),
            pltpu.SemaphoreType.DMA((4,)),
        ],
        compiler_params=pltpu.CompilerParams(collective_id=0),
    )(x, Wq_my, K_ext, V_ext, Wo_my)


# device time: 30632 ns/iter; 4.0580x vs baseline; 1.2415x over previous
import jax
import jax.numpy as jnp
from jax import lax
from jax.experimental import pallas as pl
from jax.experimental.pallas import tpu as pltpu

N_DEV = 16
B, SQ, SKV, HQ_LOC, DH = 2, 128, 128, 4, 64
D_MODEL = 512
D_HEADS = HQ_LOC * DH
ROWS = B * SQ
QROWS = ROWS // 4
ZROWS = QROWS // 4


def kernel(x, Wq, K_ext, V_ext, Wo):
    my = lax.axis_index("i")
    Wq_my = lax.dynamic_slice_in_dim(Wq, my * D_HEADS, D_HEADS, axis=1)
    Wo_my = lax.dynamic_slice_in_dim(Wo, my * D_HEADS, D_HEADS, axis=0)

    def body(x_ref, wq_ref, k_ref, v_ref, wo_ref, out_ref,
             acc_ref, pslab_ref, zslab_ref,
             prs_send, prs_recv, zrs_send, zrs_recv,
             zag_send, zag_recv, pag_send, pag_recv):
        my_pos = lax.axis_index("i")
        g = my_pos & 3
        zpos = my_pos >> 2
        base = my_pos - g

        def plane_dev(o):
            return base + (g ^ o)

        def z_dev(o):
            return ((zpos ^ o) << 2) + g

        barrier_sem = pltpu.get_barrier_semaphore()
        for o in (1, 2, 3):
            for dev in (plane_dev(o), z_dev(o)):
                pl.semaphore_signal(
                    barrier_sem, inc=1,
                    device_id=(dev,), device_id_type=pl.DeviceIdType.MESH,
                )
        pl.semaphore_wait(barrier_sem, 6)

        xm = x_ref[...].reshape(ROWS, D_MODEL)
        q = jnp.dot(xm, wq_ref[...], preferred_element_type=jnp.float32)
        q4 = q.reshape(B, SQ, HQ_LOC, DH)

        ctx_rows = []
        for bb in range(B):
            parts = []
            for h in range(HQ_LOC):
                qb = q4[bb, :, h, :]
                kb = k_ref[bb, :, h, :]
                s = lax.dot_general(
                    qb, kb, (((1,), (1,)), ((), ())),
                    preferred_element_type=jnp.float32,
                ) * 0.125
                m = jnp.max(s, axis=-1, keepdims=True)
                w = jnp.exp(s - m)
                w = w / jnp.sum(w, axis=-1, keepdims=True)
                vb = v_ref[bb, :, h, :]
                parts.append(
                    jnp.dot(w, vb, preferred_element_type=jnp.float32)
                )
            ctx_rows.append(jnp.concatenate(parts, axis=-1))
        ctx = jnp.concatenate(ctx_rows, axis=0)

        acc_ref[...] = jnp.dot(
            ctx, wo_ref[...], preferred_element_type=jnp.float32
        )

        keep_lo = g * QROWS
        blk_lo = keep_lo + zpos * ZROWS

        prs = []
        for o in (1, 2, 3):
            rdma = pltpu.make_async_remote_copy(
                src_ref=acc_ref.at[pl.ds((g ^ o) * QROWS, QROWS), :],
                dst_ref=pslab_ref.at[o - 1],
                send_sem=prs_send.at[o - 1],
                recv_sem=prs_recv.at[o - 1],
                device_id=(plane_dev(o),),
                device_id_type=pl.DeviceIdType.MESH,
            )
            rdma.start()
            prs.append(rdma)
        for rdma in prs:
            rdma.wait_recv()
        acc_ref[pl.ds(keep_lo, QROWS), :] = (
            acc_ref[pl.ds(keep_lo, QROWS), :]
            + pslab_ref[0] + pslab_ref[1] + pslab_ref[2]
        )
        for rdma in prs:
            rdma.wait_send()

        zrs = []
        for o in (1, 2, 3):
            rdma = pltpu.make_async_remote_copy(
                src_ref=acc_ref.at[pl.ds(keep_lo + (zpos ^ o) * ZROWS, ZROWS), :],
                dst_ref=zslab_ref.at[o - 1],
                send_sem=zrs_send.at[o - 1],
                recv_sem=zrs_recv.at[o - 1],
                device_id=(z_dev(o),),
                device_id_type=pl.DeviceIdType.MESH,
            )
            rdma.start()
            zrs.append(rdma)
        for rdma in zrs:
            rdma.wait_recv()
        acc_ref[pl.ds(blk_lo, ZROWS), :] = (
            acc_ref[pl.ds(blk_lo, ZROWS), :]
            + zslab_ref[0] + zslab_ref[1] + zslab_ref[2]
        )
        for rdma in zrs:
            rdma.wait_send()

        zag = []
        for o in (1, 2, 3):
            rdma = pltpu.make_async_remote_copy(
                src_ref=acc_ref.at[pl.ds(blk_lo, ZROWS), :],
                dst_ref=acc_ref.at[pl.ds(blk_lo, ZROWS), :],
                send_sem=zag_send.at[o - 1],
                recv_sem=zag_recv.at[o - 1],
                device_id=(z_dev(o),),
                device_id_type=pl.DeviceIdType.MESH,
            )
            rdma.start()
            zag.append(rdma)
        for rdma in zag:
            rdma.wait_recv()
            rdma.wait_send()

        pag = []
        for o in (1, 2, 3):
            rdma = pltpu.make_async_remote_copy(
                src_ref=acc_ref.at[pl.ds(keep_lo, QROWS), :],
                dst_ref=acc_ref.at[pl.ds(keep_lo, QROWS), :],
                send_sem=pag_send.at[o - 1],
                recv_sem=pag_recv.at[o - 1],
                device_id=(plane_dev(o),),
                device_id_type=pl.DeviceIdType.MESH,
            )
            rdma.start()
            pag.append(rdma)
        for rdma in pag:
            rdma.wait_recv()
            rdma.wait_send()

        out_ref[...] = acc_ref[...].reshape(B, SQ, D_MODEL)

    return pl.pallas_call(
        body,
        out_shape=jax.ShapeDtypeStruct((B, SQ, D_MODEL), jnp.float32),
        in_specs=[
            pl.BlockSpec(memory_space=pltpu.VMEM),
            pl.BlockSpec(memory_space=pltpu.VMEM),
            pl.BlockSpec(memory_space=pltpu.VMEM),
            pl.BlockSpec(memory_space=pltpu.VMEM),
            pl.BlockSpec(memory_space=pltpu.VMEM),
        ],
        out_specs=pl.BlockSpec(memory_space=pltpu.VMEM),
        scratch_shapes=[
            pltpu.VMEM((ROWS, D_MODEL), jnp.float32),
            pltpu.VMEM((3, QROWS, D_MODEL), jnp.float32),
            pltpu.VMEM((3, ZROWS, D_MODEL), jnp.float32),
            pltpu.SemaphoreType.DMA((3,)),
            pltpu.SemaphoreType.DMA((3,)),
            pltpu.SemaphoreType.DMA((3,)),
            pltpu.SemaphoreType.DMA((3,)),
            pltpu.SemaphoreType.DMA((3,)),
            pltpu.SemaphoreType.DMA((3,)),
            pltpu.SemaphoreType.DMA((3,)),
            pltpu.SemaphoreType.DMA((3,)),
        ],
        compiler_params=pltpu.CompilerParams(collective_id=0),
    )(x, Wq_my, K_ext, V_ext, Wo_my)
